# Initial kernel scaffold; baseline (speedup 1.0000x reference)
import jax
import jax.numpy as jnp
from jax.experimental import pallas as pl


def kernel(x, edge_index, W, b):
    raise NotImplementedError("write your pallas kernel here")


# speedup vs baseline: 3.4611x; 3.4611x over previous
import functools

import jax
import jax.numpy as jnp
from jax import lax
from jax.experimental import pallas as pl
from jax.experimental.pallas import tpu as pltpu
from jax.experimental.pallas import tpu_sc as plsc

N = 10000
NP = 10240
E = 320000
EP = 327680
D = 128
NC = 2
NS = 16
RT = 80
SL = NP // NS

_MESH = plsc.VectorSubcoreMesh(core_axis_name="c", subcore_axis_name="s")


def _fill_1d(ref, n16, value):
    def body(k, _):
        ref[pl.ds(k * 16, 16)] = jnp.full((16,), value, jnp.float32)
        return 0
    lax.fori_loop(0, n16, body, 0)


def _deg_body(src_hbm, dst_hbm, degp_hbm, src_v, dst_v, ones_v, zero_v,
              dsrc_s, ddst_s):
    c = lax.axis_index("c")
    s = lax.axis_index("s")
    _fill_1d(ones_v, 8, 1.0)
    _fill_1d(zero_v, SL // 16, 0.0)
    pltpu.sync_copy(zero_v, dsrc_s.at[pl.ds(s * SL, SL)])
    pltpu.sync_copy(zero_v, ddst_s.at[pl.ds(s * SL, SL)])
    plsc.subcore_barrier()
    base = c * (EP // D // NC) + s * (EP // D // NC // NS)
    pltpu.sync_copy(src_hbm.at[pl.ds(base, RT)], src_v)
    pltpu.sync_copy(dst_hbm.at[pl.ds(base, RT)], dst_v)

    def step(j, _):
        pltpu.sync_copy(ones_v, dsrc_s.at[src_v.at[j]], add=True)
        pltpu.sync_copy(ones_v, ddst_s.at[dst_v.at[j]], add=True)
        return 0
    lax.fori_loop(0, RT, step, 0)
    plsc.subcore_barrier()
    pltpu.sync_copy(dsrc_s.at[pl.ds(s * SL, SL)],
                    degp_hbm.at[c, 0, pl.ds(s * SL, SL)])
    pltpu.sync_copy(ddst_s.at[pl.ds(s * SL, SL)],
                    degp_hbm.at[c, 1, pl.ds(s * SL, SL)])


_deg_kernel = functools.partial(
    pl.kernel,
    out_type=jax.ShapeDtypeStruct((NC, 2, NP), jnp.float32),
    mesh=_MESH,
    scratch_types=[
        pltpu.VMEM((RT, D), jnp.int32),
        pltpu.VMEM((RT, D), jnp.int32),
        pltpu.VMEM((D,), jnp.float32),
        pltpu.VMEM((SL,), jnp.float32),
        pltpu.VMEM_SHARED((NP,), jnp.float32),
        pltpu.VMEM_SHARED((NP,), jnp.float32),
    ],
)(_deg_body)


def _gs_body(h_hbm, src_hbm, dst_hbm, aggp_hbm, src_v, dst_v, rows_v,
             agg_s, gsem):
    c = lax.axis_index("c")
    s = lax.axis_index("s")

    def fz_rows(t, _):
        rows_v[t // 8, pl.ds((t % 8) * 16, 16)] = jnp.zeros((16,), jnp.float32)
        return 0
    lax.fori_loop(0, D * 8, fz_rows, 0)

    def fz_agg(k, _):
        pltpu.sync_copy(rows_v, agg_s.at[pl.ds(s * SL + k * D, D)])
        return 0
    lax.fori_loop(0, SL // D, fz_agg, 0)
    plsc.subcore_barrier()

    base = (c * NS + s) * RT
    pltpu.sync_copy(src_hbm.at[pl.ds(base, RT)], src_v)
    pltpu.sync_copy(dst_hbm.at[pl.ds(base, RT)], dst_v)

    def step(j, _):
        pltpu.async_copy(h_hbm.at[src_v.at[j]], rows_v, gsem).wait()
        pltpu.sync_copy(rows_v, agg_s.at[dst_v.at[j]], add=True)
        return 0
    lax.fori_loop(0, RT, step, 0)
    plsc.subcore_barrier()
    pltpu.sync_copy(agg_s.at[pl.ds(s * SL, SL)],
                    aggp_hbm.at[c, pl.ds(s * SL, SL)])


_gs_kernel = functools.partial(
    pl.kernel,
    out_type=jax.ShapeDtypeStruct((NC, NP, D), jnp.float32),
    mesh=_MESH,
    scratch_types=[
        pltpu.VMEM((RT, D), jnp.int32),
        pltpu.VMEM((RT, D), jnp.int32),
        pltpu.VMEM((D, D), jnp.float32),
        pltpu.VMEM_SHARED((NP, D), jnp.float32),
        pltpu.SemaphoreType.DMA,
    ],
)(_gs_body)


def _h_body(x_ref, dsp_ref, h_ref):
    deg = dsp_ref[:, 0:1] + dsp_ref[:, 1:2]
    ns = jnp.where(deg > 0, lax.rsqrt(deg), 1.0)
    h_ref[...] = x_ref[...] * ns


def _finish_body(aggp_ref, ddp_ref, w_ref, b_ref, out_ref):
    p = aggp_ref[0] + aggp_ref[1]
    deg = ddp_ref[:, 0:1] + ddp_ref[:, 1:2]
    nd = jnp.where(deg > 0, lax.rsqrt(deg), 1.0)
    agg = p * nd
    out_ref[...] = (
        jnp.dot(agg, w_ref[...], preferred_element_type=jnp.float32)
        + b_ref[...]
    )


def kernel(x, edge_index, W, b):
    src = edge_index[0]
    dst = edge_index[1]
    pad = jnp.full((EP - E,), NP - 1, dtype=jnp.int32)
    src2 = jnp.concatenate([src, pad]).reshape(EP // D, D)
    dst2 = jnp.concatenate([dst, pad]).reshape(EP // D, D)
    x_pad = jnp.pad(x, ((0, NP - N), (0, 0)))

    degp = _deg_kernel(src2, dst2)
    dsp = degp[:, 0, :].T
    ddp = degp[:, 1, :].T

    h = pl.pallas_call(
        _h_body,
        grid=(NP // 256,),
        in_specs=[
            pl.BlockSpec((256, D), lambda i: (i, 0)),
            pl.BlockSpec((256, 2), lambda i: (i, 0)),
        ],
        out_specs=pl.BlockSpec((256, D), lambda i: (i, 0)),
        out_shape=jax.ShapeDtypeStruct((NP, D), jnp.float32),
    )(x_pad, dsp)

    aggp = _gs_kernel(h, src2, dst2)

    out = pl.pallas_call(
        _finish_body,
        grid=(N // 400,),
        in_specs=[
            pl.BlockSpec((NC, 400, D), lambda i: (0, i, 0)),
            pl.BlockSpec((400, 2), lambda i: (i, 0)),
            pl.BlockSpec((D, D), lambda i: (0, 0)),
            pl.BlockSpec((1, D), lambda i: (0, 0)),
        ],
        out_specs=pl.BlockSpec((400, D), lambda i: (i, 0)),
        out_shape=jax.ShapeDtypeStruct((N, D), jnp.float32),
    )(aggp, ddp, W, b.reshape(1, D))
    return out


# speedup vs baseline: 6.0494x; 1.7478x over previous
import functools

import jax
import jax.numpy as jnp
from jax import lax
from jax.experimental import pallas as pl
from jax.experimental.pallas import tpu as pltpu
from jax.experimental.pallas import tpu_sc as plsc

N = 10000
NP = 10240
E = 320000
EP = 327680
D = 128
DH = D // 2
NC = 2
NS = 16
ER = EP // D
RT1 = ER // NC // NS
RT3 = ER // NS
SL = NP // NS

NBUF = 5
NGIF = 3
NSIF = 2

_MESH = plsc.VectorSubcoreMesh(core_axis_name="c", subcore_axis_name="s")


def _fill_1d(ref, n16, value):
    def body(k, _):
        ref[pl.ds(k * 16, 16)] = jnp.full((16,), value, jnp.float32)
        return 0
    lax.fori_loop(0, n16, body, 0)


def _deg_body(src_hbm, dst_hbm, degp_hbm, src_v, dst_v, ones_v, zero_v,
              dsrc_s, ddst_s):
    c = lax.axis_index("c")
    s = lax.axis_index("s")
    _fill_1d(ones_v, 8, 1.0)
    _fill_1d(zero_v, SL // 16, 0.0)
    pltpu.sync_copy(zero_v, dsrc_s.at[pl.ds(s * SL, SL)])
    pltpu.sync_copy(zero_v, ddst_s.at[pl.ds(s * SL, SL)])
    plsc.subcore_barrier()
    base = (c * NS + s) * RT1
    pltpu.sync_copy(src_hbm.at[pl.ds(base, RT1)], src_v)
    pltpu.sync_copy(dst_hbm.at[pl.ds(base, RT1)], dst_v)

    def step(j, _):
        pltpu.sync_copy(ones_v, dsrc_s.at[src_v.at[j]], add=True)
        pltpu.sync_copy(ones_v, ddst_s.at[dst_v.at[j]], add=True)
        return 0
    lax.fori_loop(0, RT1, step, 0)
    plsc.subcore_barrier()
    pltpu.sync_copy(dsrc_s.at[pl.ds(s * SL, SL)],
                    degp_hbm.at[c, 0, pl.ds(s * SL, SL)])
    pltpu.sync_copy(ddst_s.at[pl.ds(s * SL, SL)],
                    degp_hbm.at[c, 1, pl.ds(s * SL, SL)])


_deg_kernel = functools.partial(
    pl.kernel,
    out_type=jax.ShapeDtypeStruct((NC, 2, NP), jnp.float32),
    mesh=_MESH,
    scratch_types=[
        pltpu.VMEM((RT1, D), jnp.int32),
        pltpu.VMEM((RT1, D), jnp.int32),
        pltpu.VMEM((D,), jnp.float32),
        pltpu.VMEM((SL,), jnp.float32),
        pltpu.VMEM_SHARED((NP,), jnp.float32),
        pltpu.VMEM_SHARED((NP,), jnp.float32),
    ],
)(_deg_body)


def _gs_body(h_hbm, srcb_hbm, dst_hbm, aggp_hbm, src_v, dst_v, rows_v,
             agg_s, gsem, ssem):
    c = lax.axis_index("c")
    s = lax.axis_index("s")

    def fz_rows(t, _):
        rows_v[0, t // 4, pl.ds((t % 4) * 16, 16)] = jnp.zeros(
            (16,), jnp.float32)
        return 0
    lax.fori_loop(0, D * 4, fz_rows, 0)

    def fz_agg(k, _):
        pltpu.sync_copy(rows_v.at[0], agg_s.at[pl.ds(s * SL + k * D, D)])
        return 0
    lax.fori_loop(0, SL // D, fz_agg, 0)
    plsc.subcore_barrier()

    pltpu.sync_copy(srcb_hbm.at[pl.ds((c * NS + s) * RT3, RT3)], src_v)
    pltpu.sync_copy(dst_hbm.at[pl.ds(s * RT3, RT3)], dst_v)

    for k in range(NGIF):
        pltpu.async_copy(h_hbm.at[src_v.at[k]], rows_v.at[k], gsem.at[k])

    def step(j, _):
        b = lax.rem(j, NBUF)
        pltpu.make_async_copy(h_hbm.at[src_v.at[j]], rows_v.at[b],
                              gsem.at[lax.rem(j, NGIF)]).wait()

        @pl.when(j >= NSIF)
        def _():
            jo = j - NSIF
            pltpu.make_async_copy(rows_v.at[lax.rem(jo, NBUF)],
                                  agg_s.at[dst_v.at[jo]],
                                  ssem.at[lax.rem(jo, NSIF)]).wait()

        pltpu.async_copy(rows_v.at[b], agg_s.at[dst_v.at[j]],
                         ssem.at[lax.rem(j, NSIF)], add=True)

        @pl.when(j + NGIF < RT3)
        def _():
            jn = j + NGIF
            pltpu.async_copy(h_hbm.at[src_v.at[jn]],
                             rows_v.at[lax.rem(jn, NBUF)],
                             gsem.at[lax.rem(jn, NGIF)])
        return 0
    lax.fori_loop(0, RT3, step, 0)
    for jj in range(RT3 - NSIF, RT3):
        pltpu.make_async_copy(rows_v.at[jj % NBUF], agg_s.at[dst_v.at[jj]],
                              ssem.at[jj % NSIF]).wait()
    plsc.subcore_barrier()
    pltpu.sync_copy(agg_s.at[pl.ds(s * SL, SL)],
                    aggp_hbm.at[c, pl.ds(s * SL, SL)])


_gs_kernel = functools.partial(
    pl.kernel,
    out_type=jax.ShapeDtypeStruct((NC, NP, DH), jnp.float32),
    mesh=_MESH,
    scratch_types=[
        pltpu.VMEM((RT3, D), jnp.int32),
        pltpu.VMEM((RT3, D), jnp.int32),
        pltpu.VMEM((NBUF, D, DH), jnp.float32),
        pltpu.VMEM_SHARED((NP, DH), jnp.float32),
        pltpu.SemaphoreType.DMA((NGIF,)),
        pltpu.SemaphoreType.DMA((NSIF,)),
    ],
    compiler_params=pltpu.CompilerParams(use_tc_tiling_on_sc=False),
)(_gs_body)


def _h_body(x_ref, dsp_ref, h_ref):
    deg = dsp_ref[:, 0:1] + dsp_ref[:, 1:2]
    ns = jnp.where(deg > 0, lax.rsqrt(deg), 1.0)
    h = x_ref[...] * ns
    h_ref[0] = h[:, :DH]
    h_ref[1] = h[:, DH:]


def _finish_body(aggp_ref, ddp_ref, w_ref, b_ref, out_ref):
    agg = jnp.concatenate([aggp_ref[0], aggp_ref[1]], axis=1)
    deg = ddp_ref[:, 0:1] + ddp_ref[:, 1:2]
    nd = jnp.where(deg > 0, lax.rsqrt(deg), 1.0)
    agg = agg * nd
    out_ref[...] = (
        jnp.dot(agg, w_ref[...], preferred_element_type=jnp.float32)
        + b_ref[...]
    )


def kernel(x, edge_index, W, b):
    src = edge_index[0]
    dst = edge_index[1]
    pad = jnp.full((EP - E,), NP - 1, dtype=jnp.int32)
    src2 = jnp.concatenate([src, pad]).reshape(ER, D)
    dst2 = jnp.concatenate([dst, pad]).reshape(ER, D)
    srcb = jnp.concatenate([src2, src2 + NP], axis=0)
    x_pad = jnp.pad(x, ((0, NP - N), (0, 0)))

    degp = _deg_kernel(src2, dst2)
    dsp = degp[:, 0, :].T
    ddp = degp[:, 1, :].T

    h2 = pl.pallas_call(
        _h_body,
        grid=(NP // 256,),
        in_specs=[
            pl.BlockSpec((256, D), lambda i: (i, 0)),
            pl.BlockSpec((256, 2), lambda i: (i, 0)),
        ],
        out_specs=pl.BlockSpec((2, 256, DH), lambda i: (0, i, 0)),
        out_shape=jax.ShapeDtypeStruct((2, NP, DH), jnp.float32),
    )(x_pad, dsp)
    h2 = h2.reshape(2 * NP, DH)

    aggp = _gs_kernel(h2, srcb, dst2)

    out = pl.pallas_call(
        _finish_body,
        grid=(N // 400,),
        in_specs=[
            pl.BlockSpec((NC, 400, DH), lambda i: (0, i, 0)),
            pl.BlockSpec((400, 2), lambda i: (i, 0)),
            pl.BlockSpec((D, D), lambda i: (0, 0)),
            pl.BlockSpec((1, D), lambda i: (0, 0)),
        ],
        out_specs=pl.BlockSpec((400, D), lambda i: (i, 0)),
        out_shape=jax.ShapeDtypeStruct((N, D), jnp.float32),
    )(aggp, ddp, W, b.reshape(1, D))
    return out


# speedup vs baseline: 6.0616x; 1.0020x over previous
import functools

import jax
import jax.numpy as jnp
from jax import lax
from jax.experimental import pallas as pl
from jax.experimental.pallas import tpu as pltpu
from jax.experimental.pallas import tpu_sc as plsc

N = 10000
NP = 10240
E = 320000
EP = 327680
D = 128
DH = D // 2
NC = 2
NS = 16
ER = EP // D
RT1 = ER // NC // NS
EC = 64
ER3 = EP // EC
RT3 = ER3 // NS
SL = NP // NS

NBUF = 10
NGIF = 6
NSIF = 4

_MESH = plsc.VectorSubcoreMesh(core_axis_name="c", subcore_axis_name="s")


def _fill_1d(ref, n16, value):
    def body(k, _):
        ref[pl.ds(k * 16, 16)] = jnp.full((16,), value, jnp.float32)
        return 0
    lax.fori_loop(0, n16, body, 0)


def _deg_body(src_hbm, dst_hbm, degp_hbm, src_v, dst_v, ones_v, zero_v,
              dsrc_s, ddst_s):
    c = lax.axis_index("c")
    s = lax.axis_index("s")
    _fill_1d(ones_v, 8, 1.0)
    _fill_1d(zero_v, SL // 16, 0.0)
    pltpu.sync_copy(zero_v, dsrc_s.at[pl.ds(s * SL, SL)])
    pltpu.sync_copy(zero_v, ddst_s.at[pl.ds(s * SL, SL)])
    plsc.subcore_barrier()
    base = (c * NS + s) * RT1
    pltpu.sync_copy(src_hbm.at[pl.ds(base, RT1)], src_v)
    pltpu.sync_copy(dst_hbm.at[pl.ds(base, RT1)], dst_v)

    def step(j, _):
        pltpu.sync_copy(ones_v, dsrc_s.at[src_v.at[j]], add=True)
        pltpu.sync_copy(ones_v, ddst_s.at[dst_v.at[j]], add=True)
        return 0
    lax.fori_loop(0, RT1, step, 0)
    plsc.subcore_barrier()
    pltpu.sync_copy(dsrc_s.at[pl.ds(s * SL, SL)],
                    degp_hbm.at[c, 0, pl.ds(s * SL, SL)])
    pltpu.sync_copy(ddst_s.at[pl.ds(s * SL, SL)],
                    degp_hbm.at[c, 1, pl.ds(s * SL, SL)])


_deg_kernel = functools.partial(
    pl.kernel,
    out_type=jax.ShapeDtypeStruct((NC, 2, NP), jnp.float32),
    mesh=_MESH,
    scratch_types=[
        pltpu.VMEM((RT1, D), jnp.int32),
        pltpu.VMEM((RT1, D), jnp.int32),
        pltpu.VMEM((D,), jnp.float32),
        pltpu.VMEM((SL,), jnp.float32),
        pltpu.VMEM_SHARED((NP,), jnp.float32),
        pltpu.VMEM_SHARED((NP,), jnp.float32),
    ],
)(_deg_body)


def _gs_body(h_hbm, srcb_hbm, dst_hbm, aggp_hbm, src_v, dst_v, rows_v,
             agg_s, gsem, ssem):
    c = lax.axis_index("c")
    s = lax.axis_index("s")

    def fz_rows(t, _):
        rows_v[0, t // 4, pl.ds((t % 4) * 16, 16)] = jnp.zeros(
            (16,), jnp.float32)
        return 0
    lax.fori_loop(0, EC * 4, fz_rows, 0)

    def fz_agg(k, _):
        pltpu.sync_copy(rows_v.at[0], agg_s.at[pl.ds(s * SL + k * EC, EC)])
        return 0
    lax.fori_loop(0, SL // EC, fz_agg, 0)
    plsc.subcore_barrier()

    pltpu.sync_copy(srcb_hbm.at[pl.ds((c * NS + s) * RT3, RT3)], src_v)
    pltpu.sync_copy(dst_hbm.at[pl.ds(s * RT3, RT3)], dst_v)

    for k in range(NGIF):
        pltpu.async_copy(h_hbm.at[src_v.at[k]], rows_v.at[k], gsem.at[k])

    def step(j, _):
        b = lax.rem(j, NBUF)
        pltpu.make_async_copy(h_hbm.at[src_v.at[j]], rows_v.at[b],
                              gsem.at[lax.rem(j, NGIF)]).wait()

        @pl.when(j >= NSIF)
        def _():
            jo = j - NSIF
            pltpu.make_async_copy(rows_v.at[lax.rem(jo, NBUF)],
                                  agg_s.at[dst_v.at[jo]],
                                  ssem.at[lax.rem(jo, NSIF)]).wait()

        pltpu.async_copy(rows_v.at[b], agg_s.at[dst_v.at[j]],
                         ssem.at[lax.rem(j, NSIF)], add=True)

        @pl.when(j + NGIF < RT3)
        def _():
            jn = j + NGIF
            pltpu.async_copy(h_hbm.at[src_v.at[jn]],
                             rows_v.at[lax.rem(jn, NBUF)],
                             gsem.at[lax.rem(jn, NGIF)])
        return 0
    lax.fori_loop(0, RT3, step, 0)
    for jj in range(RT3 - NSIF, RT3):
        pltpu.make_async_copy(rows_v.at[jj % NBUF], agg_s.at[dst_v.at[jj]],
                              ssem.at[jj % NSIF]).wait()
    plsc.subcore_barrier()
    pltpu.sync_copy(agg_s.at[pl.ds(s * SL, SL)],
                    aggp_hbm.at[c, pl.ds(s * SL, SL)])


_gs_kernel = functools.partial(
    pl.kernel,
    out_type=jax.ShapeDtypeStruct((NC, NP, DH), jnp.float32),
    mesh=_MESH,
    scratch_types=[
        pltpu.VMEM((RT3, EC), jnp.int32),
        pltpu.VMEM((RT3, EC), jnp.int32),
        pltpu.VMEM((NBUF, EC, DH), jnp.float32),
        pltpu.VMEM_SHARED((NP, DH), jnp.float32),
        pltpu.SemaphoreType.DMA((NGIF,)),
        pltpu.SemaphoreType.DMA((NSIF,)),
    ],
    compiler_params=pltpu.CompilerParams(use_tc_tiling_on_sc=False),
)(_gs_body)


def _h_body(x_ref, dsp_ref, h_ref):
    deg = dsp_ref[:, 0:1] + dsp_ref[:, 1:2]
    ns = jnp.where(deg > 0, lax.rsqrt(deg), 1.0)
    h = x_ref[...] * ns
    h_ref[0] = h[:, :DH]
    h_ref[1] = h[:, DH:]


def _finish_body(aggp_ref, ddp_ref, w_ref, b_ref, out_ref):
    agg = jnp.concatenate([aggp_ref[0], aggp_ref[1]], axis=1)
    deg = ddp_ref[:, 0:1] + ddp_ref[:, 1:2]
    nd = jnp.where(deg > 0, lax.rsqrt(deg), 1.0)
    agg = agg * nd
    out_ref[...] = (
        jnp.dot(agg, w_ref[...], preferred_element_type=jnp.float32)
        + b_ref[...]
    )


def kernel(x, edge_index, W, b):
    src = edge_index[0]
    dst = edge_index[1]
    pad = jnp.full((EP - E,), NP - 1, dtype=jnp.int32)
    srcf = jnp.concatenate([src, pad])
    dstf = jnp.concatenate([dst, pad])
    src2 = srcf.reshape(ER, D)
    dst2 = dstf.reshape(ER, D)
    srcb = jnp.concatenate([srcf, srcf + NP]).reshape(2 * ER3, EC)
    dst3 = dstf.reshape(ER3, EC)
    x_pad = jnp.pad(x, ((0, NP - N), (0, 0)))

    degp = _deg_kernel(src2, dst2)
    dsp = degp[:, 0, :].T
    ddp = degp[:, 1, :].T

    h2 = pl.pallas_call(
        _h_body,
        grid=(NP // 256,),
        in_specs=[
            pl.BlockSpec((256, D), lambda i: (i, 0)),
            pl.BlockSpec((256, 2), lambda i: (i, 0)),
        ],
        out_specs=pl.BlockSpec((2, 256, DH), lambda i: (0, i, 0)),
        out_shape=jax.ShapeDtypeStruct((2, NP, DH), jnp.float32),
    )(x_pad, dsp)
    h2 = h2.reshape(2 * NP, DH)

    aggp = _gs_kernel(h2, srcb, dst3)

    out = pl.pallas_call(
        _finish_body,
        grid=(N // 400,),
        in_specs=[
            pl.BlockSpec((NC, 400, DH), lambda i: (0, i, 0)),
            pl.BlockSpec((400, 2), lambda i: (i, 0)),
            pl.BlockSpec((D, D), lambda i: (0, 0)),
            pl.BlockSpec((1, D), lambda i: (0, 0)),
        ],
        out_specs=pl.BlockSpec((400, D), lambda i: (i, 0)),
        out_shape=jax.ShapeDtypeStruct((N, D), jnp.float32),
    )(aggp, ddp, W, b.reshape(1, D))
    return out
